# Initial kernel scaffold; baseline (speedup 1.0000x reference)
import functools

import jax
import jax.numpy as jnp
from jax import lax
from jax.experimental import pallas as pl
from jax.experimental.pallas import tpu as pltpu
import jax.experimental.pallas.tpu_sc as plsc

N_NODES = 10000
N_EDGES = 320000
D_NODE = 128
D_EDGE = 16
D_OUT = 128

NC = 2
NS = 16
NW = NC * NS
EPW = N_EDGES // NW
CH = 80
NCHUNK = EPW // CH
ROWS_PT = N_NODES // NS
LANES = 16
GRP = D_OUT // LANES


def _g_body(x_ref, wx_ref, g_ref):
    g_ref[...] = lax.dot_general(
        x_ref[...], wx_ref[...], (((1,), (1,)), ((), ())),
        preferred_element_type=jnp.float32)


def _a_body(ea_ref, we_ref, b_ref, a_ref):
    a_ref[...] = lax.dot_general(
        ea_ref[...], we_ref[...], (((1,), (1,)), ((), ())),
        preferred_element_type=jnp.float32) + b_ref[...]


def _sc_body(g_hbm, src_hbm, dst_hbm, a_hbm, zero_hbm, out_hbm,
             sidx, didx, rows, msg, acc, sem):
    cid = lax.axis_index("c")
    sid = lax.axis_index("s")
    wid = sid * NC + cid

    pltpu.sync_copy(zero_hbm.at[pl.ds(sid * ROWS_PT, ROWS_PT)],
                    acc.at[pl.ds(sid * ROWS_PT, ROWS_PT)])
    plsc.subcore_barrier()

    ebase = wid * EPW

    def chunk_body(i, carry):
        base = ebase + i * CH
        pltpu.sync_copy(src_hbm.at[pl.ds(base, CH)], sidx)
        pltpu.sync_copy(dst_hbm.at[pl.ds(base, CH)], didx)
        pltpu.async_copy(g_hbm.at[sidx], rows, sem).wait()
        pltpu.sync_copy(a_hbm.at[pl.ds(base, CH)], msg)

        def edge_body(e, c2):
            for j in range(GRP):
                s = pl.ds(j * LANES, LANES)
                msg[e, s] = jnp.maximum(rows[e, s] + msg[e, s], 0.0)
            return c2

        lax.fori_loop(0, CH, edge_body, 0)
        pltpu.sync_copy(msg, acc.at[didx], add=True)
        return carry

    lax.fori_loop(0, NCHUNK, chunk_body, 0)
    plsc.subcore_barrier()
    pltpu.sync_copy(acc.at[pl.ds(sid * ROWS_PT, ROWS_PT)],
                    out_hbm.at[cid, pl.ds(sid * ROWS_PT, ROWS_PT)])


def _combine_body(p_ref, o_ref):
    o_ref[...] = p_ref[0] + p_ref[1]


def kernel(x, edge_index, edge_attr, W, b):
    src = edge_index[0].astype(jnp.int32)
    dst = edge_index[1].astype(jnp.int32)
    We = W[:, :D_EDGE]
    Wx = W[:, D_EDGE:]
    b2 = b.reshape(1, D_OUT)
    zeros = jnp.zeros((N_NODES, D_OUT), jnp.float32)

    g = pl.pallas_call(
        _g_body,
        out_shape=jax.ShapeDtypeStruct((N_NODES, D_NODE), jnp.float32),
        grid=(4,),
        in_specs=[
            pl.BlockSpec((N_NODES // 4, D_NODE), lambda i: (i, 0)),
            pl.BlockSpec((D_OUT, D_NODE), lambda i: (0, 0)),
        ],
        out_specs=pl.BlockSpec((N_NODES // 4, D_NODE), lambda i: (i, 0)),
    )(x, Wx)

    BLK_E = 6400
    a = pl.pallas_call(
        _a_body,
        out_shape=jax.ShapeDtypeStruct((N_EDGES, D_OUT), jnp.float32),
        grid=(N_EDGES // BLK_E,),
        in_specs=[
            pl.BlockSpec((BLK_E, D_EDGE), lambda i: (i, 0)),
            pl.BlockSpec((D_OUT, D_EDGE), lambda i: (0, 0)),
            pl.BlockSpec((1, D_OUT), lambda i: (0, 0)),
        ],
        out_specs=pl.BlockSpec((BLK_E, D_OUT), lambda i: (i, 0)),
    )(edge_attr, We, b2)

    sc_call = pl.kernel(
        _sc_body,
        out_type=jax.ShapeDtypeStruct((NC, N_NODES, D_OUT), jnp.float32),
        mesh=plsc.VectorSubcoreMesh(core_axis_name="c", subcore_axis_name="s"),
        scratch_types=[
            pltpu.VMEM((CH,), jnp.int32),
            pltpu.VMEM((CH,), jnp.int32),
            pltpu.VMEM((CH, D_OUT), jnp.float32),
            pltpu.VMEM((CH, D_OUT), jnp.float32),
            pltpu.VMEM_SHARED((N_NODES, D_OUT), jnp.float32),
            pltpu.SemaphoreType.DMA,
        ],
    )
    partials = sc_call(g, src, dst, a, zeros)

    out = pl.pallas_call(
        _combine_body,
        out_shape=jax.ShapeDtypeStruct((N_NODES, D_OUT), jnp.float32),
        grid=(4,),
        in_specs=[pl.BlockSpec((NC, N_NODES // 4, D_OUT), lambda i: (0, i, 0))],
        out_specs=pl.BlockSpec((N_NODES // 4, D_OUT), lambda i: (i, 0)),
    )(partials)
    return out


# speedup vs baseline: 2.7110x; 2.7110x over previous
import functools

import jax
import jax.numpy as jnp
from jax import lax
from jax.experimental import pallas as pl
from jax.experimental.pallas import tpu as pltpu
import jax.experimental.pallas.tpu_sc as plsc

N_NODES = 10000
N_EDGES = 320000
D_NODE = 128
D_EDGE = 16
D_OUT = 128

NC = 2
NS = 16
NW = NC * NS
EPW = N_EDGES // NW
CH = 80
NCHUNK = EPW // CH
N_PAD = 10240
ROWS_PT = N_PAD // NS
LANES = 16
GRP = D_OUT // LANES


def _g_body(x_ref, wx_ref, g_ref):
    g_ref[...] = lax.dot_general(
        x_ref[...], wx_ref[...], (((1,), (1,)), ((), ())),
        preferred_element_type=jnp.float32)


def _a_body(ea_ref, we_ref, b_ref, a_ref):
    a_ref[...] = lax.dot_general(
        ea_ref[...], we_ref[...], (((1,), (1,)), ((), ())),
        preferred_element_type=jnp.float32) + b_ref[...]


def _sc_body(g_hbm, src_hbm, dst_hbm, a_hbm, zero_hbm, out_hbm,
             sidx, didx, rows, msg, acc, sem):
    cid = lax.axis_index("c")
    sid = lax.axis_index("s")
    wid = sid * NC + cid

    pltpu.sync_copy(zero_hbm.at[pl.ds(sid * ROWS_PT, ROWS_PT)],
                    acc.at[pl.ds(sid * ROWS_PT, ROWS_PT)])
    plsc.subcore_barrier()

    ebase = wid * EPW

    def chunk_body(i, carry):
        base = ebase + i * CH
        pltpu.sync_copy(src_hbm.at[pl.ds(base, CH)], sidx)
        pltpu.sync_copy(dst_hbm.at[pl.ds(base, CH)], didx)
        pltpu.async_copy(g_hbm.at[sidx], rows, sem).wait()
        pltpu.sync_copy(a_hbm.at[pl.ds(base, CH)], msg)

        def edge_body(e, c2):
            for j in range(GRP):
                s = pl.ds(j * LANES, LANES)
                msg[e, s] = jnp.maximum(rows[e, s] + msg[e, s], 0.0)
            return c2

        lax.fori_loop(0, CH, edge_body, 0)
        pltpu.sync_copy(msg, acc.at[didx], add=True)
        return carry

    lax.fori_loop(0, NCHUNK, chunk_body, 0)
    plsc.subcore_barrier()
    pltpu.sync_copy(acc.at[pl.ds(sid * ROWS_PT, ROWS_PT)],
                    out_hbm.at[cid, pl.ds(sid * ROWS_PT, ROWS_PT)])


def _combine_body(p_ref, o_ref):
    o_ref[...] = p_ref[0] + p_ref[1]


def kernel(x, edge_index, edge_attr, W, b):
    src = edge_index[0].astype(jnp.int32)
    dst = edge_index[1].astype(jnp.int32)
    We = W[:, :D_EDGE]
    Wx = W[:, D_EDGE:]
    b2 = b.reshape(1, D_OUT)
    zeros = jnp.zeros((N_PAD, D_OUT), jnp.float32)

    g = pl.pallas_call(
        _g_body,
        out_shape=jax.ShapeDtypeStruct((N_NODES, D_NODE), jnp.float32),
        grid=(5,),
        in_specs=[
            pl.BlockSpec((N_NODES // 5, D_NODE), lambda i: (i, 0)),
            pl.BlockSpec((D_OUT, D_NODE), lambda i: (0, 0)),
        ],
        out_specs=pl.BlockSpec((N_NODES // 5, D_NODE), lambda i: (i, 0)),
    )(x, Wx)

    BLK_E = 6400
    a = pl.pallas_call(
        _a_body,
        out_shape=jax.ShapeDtypeStruct((N_EDGES, D_OUT), jnp.float32),
        grid=(N_EDGES // BLK_E,),
        in_specs=[
            pl.BlockSpec((BLK_E, D_EDGE), lambda i: (i, 0)),
            pl.BlockSpec((D_OUT, D_EDGE), lambda i: (0, 0)),
            pl.BlockSpec((1, D_OUT), lambda i: (0, 0)),
        ],
        out_specs=pl.BlockSpec((BLK_E, D_OUT), lambda i: (i, 0)),
    )(edge_attr, We, b2)

    sc_call = pl.kernel(
        _sc_body,
        out_type=jax.ShapeDtypeStruct((NC, N_PAD, D_OUT), jnp.float32),
        mesh=plsc.VectorSubcoreMesh(core_axis_name="c", subcore_axis_name="s"),
        scratch_types=[
            pltpu.VMEM((CH,), jnp.int32),
            pltpu.VMEM((CH,), jnp.int32),
            pltpu.VMEM((CH, D_OUT), jnp.float32),
            pltpu.VMEM((CH, D_OUT), jnp.float32),
            pltpu.VMEM_SHARED((N_PAD, D_OUT), jnp.float32),
            pltpu.SemaphoreType.DMA,
        ],
    )
    partials = sc_call(g, src, dst, a, zeros)

    out = pl.pallas_call(
        _combine_body,
        out_shape=jax.ShapeDtypeStruct((N_PAD, D_OUT), jnp.float32),
        grid=(8,),
        in_specs=[pl.BlockSpec((NC, N_PAD // 8, D_OUT), lambda i: (0, i, 0))],
        out_specs=pl.BlockSpec((N_PAD // 8, D_OUT), lambda i: (i, 0)),
    )(partials)
    return out[:N_NODES]
